# baseline (speedup 1.0000x reference)
import functools

import jax
import jax.numpy as jnp
from jax import lax
from jax.experimental import pallas as pl
from jax.experimental.pallas import tpu as pltpu
from jax.experimental.pallas import tpu_sc as plsc

_NF = 26
_DEPTH = 1000
_W = _NF * _DEPTH
_BATCH = 4096
_NW = 32
_RPW = _BATCH // _NW
_NP = _RPW // 2


@functools.partial(
    pl.kernel,
    out_type=jax.ShapeDtypeStruct((_BATCH, _W), jnp.float32),
    mesh=plsc.VectorSubcoreMesh(core_axis_name="c", subcore_axis_name="s"),
    scratch_types=[
        pltpu.VMEM((_RPW, 32), jnp.int32),
        pltpu.VMEM((2, 2, _W), jnp.float32),
        pltpu.SemaphoreType.DMA((2,)),
    ],
    compiler_params=pltpu.CompilerParams(needs_layout_passes=False),
)
def _sc_onehot(fv_hbm, out_hbm, fv_v, row_v, sems):
    wid = lax.axis_index("s") * 2 + lax.axis_index("c")
    base = wid * _RPW
    pltpu.sync_copy(fv_hbm.at[pl.ds(base, _RPW)], fv_v)

    zeros16 = jnp.zeros((16,), jnp.float32)
    ones16 = jnp.ones((16,), jnp.float32)
    r0vec = jnp.zeros((16,), jnp.int32)
    r1vec = jnp.ones((16,), jnp.int32)

    def zbody(i, carry):
        row_v[0, 0, pl.ds(i * 16, 16)] = zeros16
        row_v[0, 1, pl.ds(i * 16, 16)] = zeros16
        row_v[1, 0, pl.ds(i * 16, 16)] = zeros16
        row_v[1, 1, pl.ds(i * 16, 16)] = zeros16
        return carry

    lax.fori_loop(0, _W // 16, zbody, 0)

    iota = lax.iota(jnp.int32, 16)
    off0 = iota * _DEPTH
    off1 = (iota + 16) * _DEPTH
    mask1 = iota < (_NF - 16)

    def pair_pos(k):
        p00 = fv_v[2 * k, pl.ds(0, 16)] + off0
        p01 = fv_v[2 * k, pl.ds(16, 16)] + off1
        p10 = fv_v[2 * k + 1, pl.ds(0, 16)] + off0
        p11 = fv_v[2 * k + 1, pl.ds(16, 16)] + off1
        return p00, p01, p10, p11

    def scatter_pair(slot, k, val):
        p00, p01, p10, p11 = pair_pos(k)
        buf = row_v.at[slot]
        plsc.store_scatter(buf, [r0vec, p00], val)
        plsc.store_scatter(buf, [r0vec, p01], val, mask=mask1)
        plsc.store_scatter(buf, [r1vec, p10], val)
        plsc.store_scatter(buf, [r1vec, p11], val, mask=mask1)

    def copy_desc(slot, k):
        return pltpu.make_async_copy(
            row_v.at[slot], out_hbm.at[pl.ds(base + 2 * k, 2)],
            sems.at[slot])

    def kbody(k, carry):
        slot = lax.rem(k, 2)

        @pl.when(k >= 2)
        def _retire():
            copy_desc(slot, k - 2).wait()
            scatter_pair(slot, k - 2, zeros16)

        scatter_pair(slot, k, ones16)
        copy_desc(slot, k).start()
        return carry

    lax.fori_loop(0, _NP, kbody, 0)
    for t in (0, 1):
        kk = _NP - 2 + t
        copy_desc(kk & 1, kk).wait()


def kernel(feature_value):
    fv_pad = jnp.pad(feature_value, ((0, 0), (0, 32 - _NF)))
    return _sc_onehot(fv_pad)


# speedup vs baseline: 3.9999x; 3.9999x over previous
import jax
import jax.numpy as jnp
from jax.experimental import pallas as pl

_NUM_FIELDS = 26
_DEPTH = 1000


def _onehot_t_block(fvt_ref, out_ref):
    fv_row = fvt_ref[0]
    pos = jax.lax.broadcasted_iota(jnp.int32, out_ref.shape, 0)
    out_ref[...] = (pos == fv_row).astype(jnp.float32)


def kernel(feature_value):
    batch = feature_value.shape[0]
    fvt = feature_value.T.reshape(_NUM_FIELDS, 1, batch)
    out_t = pl.pallas_call(
        _onehot_t_block,
        grid=(_NUM_FIELDS,),
        in_specs=[pl.BlockSpec((1, 1, batch), lambda f: (f, 0, 0))],
        out_specs=pl.BlockSpec((_DEPTH, batch), lambda f: (f, 0)),
        out_shape=jax.ShapeDtypeStruct((_NUM_FIELDS * _DEPTH, batch),
                                       jnp.float32),
    )(fvt)
    return out_t.T
